# baseline (device time: 25455 ns/iter reference)
import jax
import jax.numpy as jnp
from jax import lax
from jax.experimental import pallas as pl
from jax.experimental.pallas import tpu as pltpu

N_DEV = 32


def kernel(x, gamma, beta):
    m, n_per = x.shape
    n_total = n_per * N_DEV

    def body(x_ref, gamma_ref, beta_ref, out_ref, gather_ref, send_sems, recv_sems):
        my = lax.axis_index("i")

        barrier_sem = pltpu.get_barrier_semaphore()
        for d in range(1, N_DEV):
            pl.semaphore_signal(
                barrier_sem,
                inc=1,
                device_id=((my + d) % N_DEV,),
                device_id_type=pl.DeviceIdType.MESH,
            )

        x_val = x_ref[:, :]
        gather_ref[0, 0, :] = jnp.sum(x_val, axis=1)
        gather_ref[0, 1, :] = jnp.sum(x_val * x_val, axis=1)

        pl.semaphore_wait(barrier_sem, N_DEV - 1)

        rdmas = []
        for d in range(1, N_DEV):
            rdma = pltpu.make_async_remote_copy(
                src_ref=gather_ref.at[0],
                dst_ref=gather_ref.at[d],
                send_sem=send_sems.at[d],
                recv_sem=recv_sems.at[d],
                device_id=((my + d) % N_DEV,),
                device_id_type=pl.DeviceIdType.MESH,
            )
            rdma.start()
            rdmas.append(rdma)
        for rdma in rdmas:
            rdma.wait()

        g = gather_ref[:, :, :]
        totals = jnp.sum(g, axis=0)
        mean = totals[0, :] / n_total
        var = totals[1, :] / n_total - mean * mean
        inv = lax.rsqrt(var + 1e-5)

        normed = (x_val - mean[:, None]) * inv[:, None]
        out_ref[:, :] = gamma_ref[:, :] * normed + beta_ref[:, :]

    return pl.pallas_call(
        body,
        out_shape=jax.ShapeDtypeStruct((m, n_per), jnp.float32),
        in_specs=[
            pl.BlockSpec(memory_space=pltpu.VMEM),
            pl.BlockSpec(memory_space=pltpu.VMEM),
            pl.BlockSpec(memory_space=pltpu.VMEM),
        ],
        out_specs=pl.BlockSpec(memory_space=pltpu.VMEM),
        scratch_shapes=[
            pltpu.VMEM((N_DEV, 2, m), jnp.float32),
            pltpu.SemaphoreType.DMA((N_DEV,)),
            pltpu.SemaphoreType.DMA((N_DEV,)),
        ],
        compiler_params=pltpu.CompilerParams(collective_id=0),
    )(x, gamma.reshape(1, n_per), beta.reshape(1, n_per))


# device time: 23962 ns/iter; 1.0623x vs baseline; 1.0623x over previous
import jax
import jax.numpy as jnp
from jax import lax
from jax.experimental import pallas as pl
from jax.experimental.pallas import tpu as pltpu

N_DEV = 32
GROUP = 8
N_GROUPS = N_DEV // GROUP


def kernel(x, gamma, beta):
    m, n_per = x.shape
    n_total = n_per * N_DEV

    def body(
        x_ref, gamma_ref, beta_ref, out_ref,
        buf1_ref, buf2_ref, s1_send, s1_recv, s2_send, s2_recv,
    ):
        my = lax.axis_index("i")
        g = my // GROUP
        idx = my % GROUP

        barrier_sem = pltpu.get_barrier_semaphore()
        for d in range(1, GROUP):
            pl.semaphore_signal(
                barrier_sem, inc=1,
                device_id=(g * GROUP + (idx + d) % GROUP,),
                device_id_type=pl.DeviceIdType.MESH,
            )
        for e in range(1, N_GROUPS):
            pl.semaphore_signal(
                barrier_sem, inc=1,
                device_id=(((g + e) % N_GROUPS) * GROUP + idx,),
                device_id_type=pl.DeviceIdType.MESH,
            )

        x_val = x_ref[:, :]
        buf1_ref[0, 0, :] = jnp.sum(x_val, axis=1)
        buf1_ref[0, 1, :] = jnp.sum(x_val * x_val, axis=1)

        pl.semaphore_wait(barrier_sem, GROUP - 1 + N_GROUPS - 1)

        st1 = []
        for d in range(1, GROUP):
            rdma = pltpu.make_async_remote_copy(
                src_ref=buf1_ref.at[0],
                dst_ref=buf1_ref.at[d],
                send_sem=s1_send.at[d],
                recv_sem=s1_recv.at[d],
                device_id=(g * GROUP + (idx + d) % GROUP,),
                device_id_type=pl.DeviceIdType.MESH,
            )
            rdma.start()
            st1.append(rdma)
        for rdma in st1:
            rdma.wait()

        buf2_ref[0, :, :] = jnp.sum(buf1_ref[:, :, :], axis=0)

        st2 = []
        for e in range(1, N_GROUPS):
            rdma = pltpu.make_async_remote_copy(
                src_ref=buf2_ref.at[0],
                dst_ref=buf2_ref.at[e],
                send_sem=s2_send.at[e],
                recv_sem=s2_recv.at[e],
                device_id=(((g + e) % N_GROUPS) * GROUP + idx,),
                device_id_type=pl.DeviceIdType.MESH,
            )
            rdma.start()
            st2.append(rdma)
        for rdma in st2:
            rdma.wait()

        totals = jnp.sum(buf2_ref[:, :, :], axis=0)
        mean = totals[0, :] / n_total
        var = totals[1, :] / n_total - mean * mean
        inv = lax.rsqrt(var + 1e-5)

        normed = (x_val - mean[:, None]) * inv[:, None]
        out_ref[:, :] = gamma_ref[:, :] * normed + beta_ref[:, :]

    return pl.pallas_call(
        body,
        out_shape=jax.ShapeDtypeStruct((m, n_per), jnp.float32),
        in_specs=[
            pl.BlockSpec(memory_space=pltpu.VMEM),
            pl.BlockSpec(memory_space=pltpu.VMEM),
            pl.BlockSpec(memory_space=pltpu.VMEM),
        ],
        out_specs=pl.BlockSpec(memory_space=pltpu.VMEM),
        scratch_shapes=[
            pltpu.VMEM((GROUP, 2, m), jnp.float32),
            pltpu.VMEM((N_GROUPS, 2, m), jnp.float32),
            pltpu.SemaphoreType.DMA((GROUP,)),
            pltpu.SemaphoreType.DMA((GROUP,)),
            pltpu.SemaphoreType.DMA((N_GROUPS,)),
            pltpu.SemaphoreType.DMA((N_GROUPS,)),
        ],
        compiler_params=pltpu.CompilerParams(collective_id=0),
    )(x, gamma.reshape(1, n_per), beta.reshape(1, n_per))
